# baseline (speedup 1.0000x reference)
import jax
import jax.numpy as jnp
from jax.experimental import pallas as pl
from jax.experimental.pallas import tpu as pltpu

EOS = 1e-10
BM = 1024
BN = 1024


def _elu1(a):
    return jnp.where(a > 0, a + 1.0, jnp.exp(a))


def _pair_maps(nb):
    im, jm = [], []
    for i in range(nb):
        for j in range(i, nb):
            im.append(i)
            jm.append(j)
    return jnp.array(im, jnp.int32), jnp.array(jm, jnp.int32)


def _stats_kernel(a_ref, x_ref, w1_ref, b1_ref, rowsum_ref, colpart_ref, h1_ref):
    j = pl.program_id(1)
    e = _elu1(a_ref[:])
    rs = jnp.sum(e, axis=1, keepdims=True)

    @pl.when(j == 0)
    def _():
        rowsum_ref[:] = rs
        h1_ref[:] = (
            jnp.dot(x_ref[:], w1_ref[:], preferred_element_type=jnp.float32)
            + b1_ref[:]
        )

    @pl.when(j != 0)
    def _():
        rowsum_ref[:] += rs

    colpart_ref[:] = jnp.sum(e, axis=0).reshape(1, 1, -1)


def _main_kernel(
    im_ref, jm_ref, a_ref, at_ref, h1j_ref, h1i_ref, dc_ref, dr_ref,
    adjn_ref, y1_ref, yp_ref, scr_ref,
):
    k = pl.program_id(0)
    s = pl.program_id(1)
    i = im_ref[k]
    j = jm_ref[k]

    @pl.when(s == 0)
    def _():
        e = 0.5 * (_elu1(a_ref[:]) + _elu1(at_ref[:]).T)
        adjn = dc_ref[:] * e * dr_ref[:]
        adjn_ref[:] = adjn
        scr_ref[:] = adjn
        c = jnp.dot(adjn, h1j_ref[:], preferred_element_type=jnp.float32)

        @pl.when(i == j)
        def _():
            y1_ref[:] = c

        @pl.when(i != j)
        def _():
            y1_ref[:] += c

    @pl.when(s == 1)
    def _():
        @pl.when(i != j)
        def _():
            adjn_ref[:] = scr_ref[:].T
            yp_ref[:] = jax.lax.dot_general(
                scr_ref[:], h1i_ref[:],
                (((0,), (0,)), ((), ())),
                preferred_element_type=jnp.float32,
            ).reshape(yp_ref.shape)

        @pl.when(i == j)
        def _():
            yp_ref[:] = jnp.zeros_like(yp_ref)


def _out_kernel(
    im_ref, jm_ref, adjn_ref, y1j_ref, w2_ref, b2_ref,
    out_ref, op_ref, h2_ref,
):
    k = pl.program_id(0)
    i = im_ref[k]
    j = jm_ref[k]

    @pl.when(i == 0)
    def _():
        h = jnp.maximum(y1j_ref[:], 0.0)
        h2_ref[pl.ds(j * BN, BN), :] = (
            jnp.dot(h, w2_ref[:], preferred_element_type=jnp.float32) + b2_ref[:]
        )

    c = jnp.dot(
        adjn_ref[:], h2_ref[pl.ds(j * BN, BN), :], preferred_element_type=jnp.float32
    )

    @pl.when(i == j)
    def _():
        out_ref[:] = c
        op_ref[:] = jnp.zeros_like(op_ref)

    @pl.when(i != j)
    def _():
        out_ref[:] += c
        op_ref[:] = jax.lax.dot_general(
            adjn_ref[:], h2_ref[pl.ds(i * BM, BM), :],
            (((0,), (0,)), ((), ())),
            preferred_element_type=jnp.float32,
        ).reshape(op_ref.shape)


def kernel(features, x, Adj_param, W1, b1, W2, b2):
    N = Adj_param.shape[0]
    in_dim = x.shape[1]
    hid = W1.shape[1]
    ncls = W2.shape[1]
    nb = N // BM
    npairs = nb * (nb + 1) // 2
    im, jm = _pair_maps(nb)

    rowsum, colpart, h1 = pl.pallas_call(
        _stats_kernel,
        grid=(nb, nb),
        in_specs=[
            pl.BlockSpec((BM, BN), lambda i, j: (i, j)),
            pl.BlockSpec((BM, in_dim), lambda i, j: (i, 0)),
            pl.BlockSpec((in_dim, hid), lambda i, j: (0, 0)),
            pl.BlockSpec((1, hid), lambda i, j: (0, 0)),
        ],
        out_specs=[
            pl.BlockSpec((BM, 1), lambda i, j: (i, 0)),
            pl.BlockSpec((1, 1, BN), lambda i, j: (i, 0, j)),
            pl.BlockSpec((BM, hid), lambda i, j: (i, 0)),
        ],
        out_shape=[
            jax.ShapeDtypeStruct((N, 1), jnp.float32),
            jax.ShapeDtypeStruct((nb, 1, N), jnp.float32),
            jax.ShapeDtypeStruct((N, hid), jnp.float32),
        ],
    )(Adj_param, x, W1, b1.reshape(1, hid))

    deg = 0.5 * (rowsum[:, 0] + jnp.sum(colpart, axis=(0, 1)))
    dinv = 1.0 / (jnp.sqrt(deg) + EOS)
    dc = dinv[:, None]
    dr = dinv[None, :]

    adjn, y1m, y1p = pl.pallas_call(
        _main_kernel,
        grid_spec=pltpu.PrefetchScalarGridSpec(
            num_scalar_prefetch=2,
            grid=(npairs, 2),
            in_specs=[
                pl.BlockSpec((BM, BN), lambda k, s, im, jm: (im[k], jm[k])),
                pl.BlockSpec((BN, BM), lambda k, s, im, jm: (jm[k], im[k])),
                pl.BlockSpec((BN, hid), lambda k, s, im, jm: (jm[k], 0)),
                pl.BlockSpec((BM, hid), lambda k, s, im, jm: (im[k], 0)),
                pl.BlockSpec((BM, 1), lambda k, s, im, jm: (im[k], 0)),
                pl.BlockSpec((1, BN), lambda k, s, im, jm: (0, jm[k])),
            ],
            out_specs=[
                pl.BlockSpec(
                    (BM, BN),
                    lambda k, s, im, jm: (
                        jnp.where(s == 0, im[k], jm[k]),
                        jnp.where(s == 0, jm[k], im[k]),
                    ),
                ),
                pl.BlockSpec((BM, hid), lambda k, s, im, jm: (im[k], 0)),
                pl.BlockSpec((1, BN, hid), lambda k, s, im, jm: (k, 0, 0)),
            ],
            scratch_shapes=[pltpu.VMEM((BM, BN), jnp.float32)],
        ),
        out_shape=[
            jax.ShapeDtypeStruct((N, N), jnp.float32),
            jax.ShapeDtypeStruct((N, hid), jnp.float32),
            jax.ShapeDtypeStruct((npairs, BN, hid), jnp.float32),
        ],
    )(im, jm, Adj_param, Adj_param, h1, h1, dc, dr)

    y1 = y1m + jax.ops.segment_sum(
        y1p, jm, num_segments=nb, indices_are_sorted=True
    ).reshape(N, hid)

    out_m, out_p = pl.pallas_call(
        _out_kernel,
        grid_spec=pltpu.PrefetchScalarGridSpec(
            num_scalar_prefetch=2,
            grid=(npairs,),
            in_specs=[
                pl.BlockSpec((BM, BN), lambda k, im, jm: (im[k], jm[k])),
                pl.BlockSpec((BN, hid), lambda k, im, jm: (jm[k], 0)),
                pl.BlockSpec((hid, ncls), lambda k, im, jm: (0, 0)),
                pl.BlockSpec((1, ncls), lambda k, im, jm: (0, 0)),
            ],
            out_specs=[
                pl.BlockSpec((BM, ncls), lambda k, im, jm: (im[k], 0)),
                pl.BlockSpec((1, BN, ncls), lambda k, im, jm: (k, 0, 0)),
            ],
            scratch_shapes=[pltpu.VMEM((N, ncls), jnp.float32)],
        ),
        out_shape=[
            jax.ShapeDtypeStruct((N, ncls), jnp.float32),
            jax.ShapeDtypeStruct((npairs, BN, ncls), jnp.float32),
        ],
    )(im, jm, adjn, y1, W2, b2.reshape(1, ncls))

    out = out_m + jax.ops.segment_sum(
        out_p, jm, num_segments=nb, indices_are_sorted=True
    ).reshape(N, ncls)

    return (out, adjn)


# speedup vs baseline: 1.1638x; 1.1638x over previous
import jax
import jax.numpy as jnp
from jax.experimental import pallas as pl
from jax.experimental.pallas import tpu as pltpu

EOS = 1e-10
BM = 1024
BN = 1024


def _elu1(a):
    return jnp.where(a > 0, a + 1.0, jnp.exp(a))


def _pair_maps(nb):
    im, jm = [], []
    for i in range(nb):
        for j in range(i, nb):
            im.append(i)
            jm.append(j)
    return jnp.array(im, jnp.int32), jnp.array(jm, jnp.int32)


def _stats_kernel(a_ref, x_ref, w1_ref, b1_ref, rowsum_ref, colpart_ref, h1_ref):
    j = pl.program_id(1)
    e = _elu1(a_ref[:])
    rs = jnp.sum(e, axis=1, keepdims=True)

    @pl.when(j == 0)
    def _():
        rowsum_ref[:] = rs
        h1_ref[:] = (
            jnp.dot(x_ref[:], w1_ref[:], preferred_element_type=jnp.float32)
            + b1_ref[:]
        )

    @pl.when(j != 0)
    def _():
        rowsum_ref[:] += rs

    colpart_ref[:] = jnp.sum(e, axis=0).reshape(1, 1, -1)


def _main_kernel(a_ref, at_ref, h1_ref, dc_ref, dr_ref, adjn_ref, y1_ref):
    j = pl.program_id(1)
    e = 0.5 * (_elu1(a_ref[:]) + _elu1(at_ref[:]).T)
    adjn = dc_ref[:] * e * dr_ref[:]
    adjn_ref[:] = adjn
    c = jnp.dot(adjn, h1_ref[:], preferred_element_type=jnp.float32)

    @pl.when(j == 0)
    def _():
        y1_ref[:] = c

    @pl.when(j != 0)
    def _():
        y1_ref[:] += c


def _out_kernel(
    im_ref, jm_ref, adjn_ref, y1j_ref, w2_ref, b2_ref,
    out_ref, op_ref, h2_ref,
):
    k = pl.program_id(0)
    i = im_ref[k]
    j = jm_ref[k]

    @pl.when(i == 0)
    def _():
        h = jnp.maximum(y1j_ref[:], 0.0)
        h2_ref[pl.ds(j * BN, BN), :] = (
            jnp.dot(h, w2_ref[:], preferred_element_type=jnp.float32) + b2_ref[:]
        )

    c = jnp.dot(
        adjn_ref[:], h2_ref[pl.ds(j * BN, BN), :], preferred_element_type=jnp.float32
    )

    @pl.when(i == j)
    def _():
        out_ref[:] = c
        op_ref[:] = jnp.zeros_like(op_ref)

    @pl.when(i != j)
    def _():
        out_ref[:] += c
        op_ref[:] = jax.lax.dot_general(
            adjn_ref[:], h2_ref[pl.ds(i * BM, BM), :],
            (((0,), (0,)), ((), ())),
            preferred_element_type=jnp.float32,
        ).reshape(op_ref.shape)


def kernel(features, x, Adj_param, W1, b1, W2, b2):
    N = Adj_param.shape[0]
    in_dim = x.shape[1]
    hid = W1.shape[1]
    ncls = W2.shape[1]
    nb = N // BM
    npairs = nb * (nb + 1) // 2
    im, jm = _pair_maps(nb)

    rowsum, colpart, h1 = pl.pallas_call(
        _stats_kernel,
        grid=(nb, nb),
        in_specs=[
            pl.BlockSpec((BM, BN), lambda i, j: (i, j)),
            pl.BlockSpec((BM, in_dim), lambda i, j: (i, 0)),
            pl.BlockSpec((in_dim, hid), lambda i, j: (0, 0)),
            pl.BlockSpec((1, hid), lambda i, j: (0, 0)),
        ],
        out_specs=[
            pl.BlockSpec((BM, 1), lambda i, j: (i, 0)),
            pl.BlockSpec((1, 1, BN), lambda i, j: (i, 0, j)),
            pl.BlockSpec((BM, hid), lambda i, j: (i, 0)),
        ],
        out_shape=[
            jax.ShapeDtypeStruct((N, 1), jnp.float32),
            jax.ShapeDtypeStruct((nb, 1, N), jnp.float32),
            jax.ShapeDtypeStruct((N, hid), jnp.float32),
        ],
    )(Adj_param, x, W1, b1.reshape(1, hid))

    deg = 0.5 * (rowsum[:, 0] + jnp.sum(colpart, axis=(0, 1)))
    dinv = 1.0 / (jnp.sqrt(deg) + EOS)
    dc = dinv[:, None]
    dr = dinv[None, :]

    adjn, y1 = pl.pallas_call(
        _main_kernel,
        grid=(nb, nb),
        in_specs=[
            pl.BlockSpec((BM, BN), lambda i, j: (i, j)),
            pl.BlockSpec((BN, BM), lambda i, j: (j, i)),
            pl.BlockSpec((BN, hid), lambda i, j: (j, 0)),
            pl.BlockSpec((BM, 1), lambda i, j: (i, 0)),
            pl.BlockSpec((1, BN), lambda i, j: (0, j)),
        ],
        out_specs=[
            pl.BlockSpec((BM, BN), lambda i, j: (i, j)),
            pl.BlockSpec((BM, hid), lambda i, j: (i, 0)),
        ],
        out_shape=[
            jax.ShapeDtypeStruct((N, N), jnp.float32),
            jax.ShapeDtypeStruct((N, hid), jnp.float32),
        ],
    )(Adj_param, Adj_param, h1, dc, dr)

    out_m, out_p = pl.pallas_call(
        _out_kernel,
        grid_spec=pltpu.PrefetchScalarGridSpec(
            num_scalar_prefetch=2,
            grid=(npairs,),
            in_specs=[
                pl.BlockSpec((BM, BN), lambda k, im, jm: (im[k], jm[k])),
                pl.BlockSpec((BN, hid), lambda k, im, jm: (jm[k], 0)),
                pl.BlockSpec((hid, ncls), lambda k, im, jm: (0, 0)),
                pl.BlockSpec((1, ncls), lambda k, im, jm: (0, 0)),
            ],
            out_specs=[
                pl.BlockSpec((BM, ncls), lambda k, im, jm: (im[k], 0)),
                pl.BlockSpec((1, BN, ncls), lambda k, im, jm: (k, 0, 0)),
            ],
            scratch_shapes=[pltpu.VMEM((N, ncls), jnp.float32)],
        ),
        out_shape=[
            jax.ShapeDtypeStruct((N, ncls), jnp.float32),
            jax.ShapeDtypeStruct((npairs, BN, ncls), jnp.float32),
        ],
    )(im, jm, adjn, y1, W2, b2.reshape(1, ncls))

    out = out_m + jax.ops.segment_sum(
        out_p, jm, num_segments=nb, indices_are_sorted=True
    ).reshape(N, ncls)

    return (out, adjn)


# speedup vs baseline: 1.2220x; 1.0500x over previous
import jax
import jax.numpy as jnp
from jax.experimental import pallas as pl
from jax.experimental.pallas import tpu as pltpu

EOS = 1e-10
BM = 1024
BN = 1024


def _elu1(a):
    return jnp.where(a > 0, a + 1.0, jnp.exp(a))


def _pair_maps(nb):
    im, jm = [], []
    for i in range(nb):
        for j in range(i, nb):
            im.append(i)
            jm.append(j)
    return jnp.array(im, jnp.int32), jnp.array(jm, jnp.int32)


def _stats_kernel(a_ref, x_ref, w1_ref, b1_ref, rowsum_ref, colpart_ref, h1_ref):
    j = pl.program_id(1)
    e = _elu1(a_ref[:])
    rs = jnp.sum(e, axis=1, keepdims=True)

    @pl.when(j == 0)
    def _():
        rowsum_ref[:] = rs
        h1_ref[:] = (
            jnp.dot(x_ref[:], w1_ref[:], preferred_element_type=jnp.float32)
            + b1_ref[:]
        )

    @pl.when(j != 0)
    def _():
        rowsum_ref[:] += rs

    colpart_ref[:] = jnp.sum(e, axis=0).reshape(1, 1, -1)


def _main_kernel(
    im_ref, jm_ref, a_ref, at_ref, h1j_ref, h1i_ref, dc_ref, dr_ref,
    adjn_ref, y1_ref, yp_ref, ab_ref, tb_ref, sema_ref, semb_ref,
):
    k = pl.program_id(0)
    i = im_ref[k]
    j = jm_ref[k]
    e = 0.5 * (_elu1(a_ref[:]) + _elu1(at_ref[:]).T)
    adjn = dc_ref[:] * e * dr_ref[:]
    ab_ref[:] = adjn
    pltpu.make_async_copy(
        ab_ref,
        adjn_ref.at[pl.ds(i * BM, BM), pl.ds(j * BN, BN)],
        sema_ref,
    ).start()
    tb_ref[:] = adjn.T
    c = jnp.dot(adjn, h1j_ref[:], preferred_element_type=jnp.float32)

    @pl.when(i == j)
    def _():
        y1_ref[:] = c
        yp_ref[:] = jnp.zeros_like(yp_ref)

    @pl.when(i != j)
    def _():
        y1_ref[:] += c
        yp_ref[:] = jax.lax.dot_general(
            adjn, h1i_ref[:],
            (((0,), (0,)), ((), ())),
            preferred_element_type=jnp.float32,
        ).reshape(yp_ref.shape)
        pltpu.make_async_copy(
            tb_ref,
            adjn_ref.at[pl.ds(j * BM, BM), pl.ds(i * BN, BN)],
            semb_ref,
        ).start()

    pltpu.make_async_copy(
        ab_ref,
        adjn_ref.at[pl.ds(i * BM, BM), pl.ds(j * BN, BN)],
        sema_ref,
    ).wait()

    @pl.when(i != j)
    def _():
        pltpu.make_async_copy(
            tb_ref,
            adjn_ref.at[pl.ds(j * BM, BM), pl.ds(i * BN, BN)],
            semb_ref,
        ).wait()


def _out_kernel(
    im_ref, jm_ref, adjn_ref, y1j_ref, w2_ref, b2_ref,
    out_ref, op_ref, h2_ref,
):
    k = pl.program_id(0)
    i = im_ref[k]
    j = jm_ref[k]

    @pl.when(i == 0)
    def _():
        h = jnp.maximum(y1j_ref[:], 0.0)
        h2_ref[pl.ds(j * BN, BN), :] = (
            jnp.dot(h, w2_ref[:], preferred_element_type=jnp.float32) + b2_ref[:]
        )

    c = jnp.dot(
        adjn_ref[:], h2_ref[pl.ds(j * BN, BN), :], preferred_element_type=jnp.float32
    )

    @pl.when(i == j)
    def _():
        out_ref[:] = c
        op_ref[:] = jnp.zeros_like(op_ref)

    @pl.when(i != j)
    def _():
        out_ref[:] += c
        op_ref[:] = jax.lax.dot_general(
            adjn_ref[:], h2_ref[pl.ds(i * BM, BM), :],
            (((0,), (0,)), ((), ())),
            preferred_element_type=jnp.float32,
        ).reshape(op_ref.shape)


def kernel(features, x, Adj_param, W1, b1, W2, b2):
    N = Adj_param.shape[0]
    in_dim = x.shape[1]
    hid = W1.shape[1]
    ncls = W2.shape[1]
    nb = N // BM
    npairs = nb * (nb + 1) // 2
    im, jm = _pair_maps(nb)

    rowsum, colpart, h1 = pl.pallas_call(
        _stats_kernel,
        grid=(nb, nb),
        in_specs=[
            pl.BlockSpec((BM, BN), lambda i, j: (i, j)),
            pl.BlockSpec((BM, in_dim), lambda i, j: (i, 0)),
            pl.BlockSpec((in_dim, hid), lambda i, j: (0, 0)),
            pl.BlockSpec((1, hid), lambda i, j: (0, 0)),
        ],
        out_specs=[
            pl.BlockSpec((BM, 1), lambda i, j: (i, 0)),
            pl.BlockSpec((1, 1, BN), lambda i, j: (i, 0, j)),
            pl.BlockSpec((BM, hid), lambda i, j: (i, 0)),
        ],
        out_shape=[
            jax.ShapeDtypeStruct((N, 1), jnp.float32),
            jax.ShapeDtypeStruct((nb, 1, N), jnp.float32),
            jax.ShapeDtypeStruct((N, hid), jnp.float32),
        ],
    )(Adj_param, x, W1, b1.reshape(1, hid))

    deg = 0.5 * (rowsum[:, 0] + jnp.sum(colpart, axis=(0, 1)))
    dinv = 1.0 / (jnp.sqrt(deg) + EOS)
    dc = dinv[:, None]
    dr = dinv[None, :]

    adjn, y1m, y1p = pl.pallas_call(
        _main_kernel,
        grid_spec=pltpu.PrefetchScalarGridSpec(
            num_scalar_prefetch=2,
            grid=(npairs,),
            in_specs=[
                pl.BlockSpec((BM, BN), lambda k, im, jm: (im[k], jm[k])),
                pl.BlockSpec((BN, BM), lambda k, im, jm: (jm[k], im[k])),
                pl.BlockSpec((BN, hid), lambda k, im, jm: (jm[k], 0)),
                pl.BlockSpec((BM, hid), lambda k, im, jm: (im[k], 0)),
                pl.BlockSpec((BM, 1), lambda k, im, jm: (im[k], 0)),
                pl.BlockSpec((1, BN), lambda k, im, jm: (0, jm[k])),
            ],
            out_specs=[
                pl.BlockSpec(memory_space=pl.ANY),
                pl.BlockSpec((BM, hid), lambda k, im, jm: (im[k], 0)),
                pl.BlockSpec((1, BN, hid), lambda k, im, jm: (k, 0, 0)),
            ],
            scratch_shapes=[
                pltpu.VMEM((BM, BN), jnp.float32),
                pltpu.VMEM((BN, BM), jnp.float32),
                pltpu.SemaphoreType.DMA,
                pltpu.SemaphoreType.DMA,
            ],
        ),
        out_shape=[
            jax.ShapeDtypeStruct((N, N), jnp.float32),
            jax.ShapeDtypeStruct((N, hid), jnp.float32),
            jax.ShapeDtypeStruct((npairs, BN, hid), jnp.float32),
        ],
    )(im, jm, Adj_param, Adj_param, h1, h1, dc, dr)

    y1 = y1m + jax.ops.segment_sum(
        y1p, jm, num_segments=nb, indices_are_sorted=True
    ).reshape(N, hid)

    out_m, out_p = pl.pallas_call(
        _out_kernel,
        grid_spec=pltpu.PrefetchScalarGridSpec(
            num_scalar_prefetch=2,
            grid=(npairs,),
            in_specs=[
                pl.BlockSpec((BM, BN), lambda k, im, jm: (im[k], jm[k])),
                pl.BlockSpec((BN, hid), lambda k, im, jm: (jm[k], 0)),
                pl.BlockSpec((hid, ncls), lambda k, im, jm: (0, 0)),
                pl.BlockSpec((1, ncls), lambda k, im, jm: (0, 0)),
            ],
            out_specs=[
                pl.BlockSpec((BM, ncls), lambda k, im, jm: (im[k], 0)),
                pl.BlockSpec((1, BN, ncls), lambda k, im, jm: (k, 0, 0)),
            ],
            scratch_shapes=[pltpu.VMEM((N, ncls), jnp.float32)],
        ),
        out_shape=[
            jax.ShapeDtypeStruct((N, ncls), jnp.float32),
            jax.ShapeDtypeStruct((npairs, BN, ncls), jnp.float32),
        ],
    )(im, jm, adjn, y1, W2, b2.reshape(1, ncls))

    out = out_m + jax.ops.segment_sum(
        out_p, jm, num_segments=nb, indices_are_sorted=True
    ).reshape(N, ncls)

    return (out, adjn)


# speedup vs baseline: 1.2438x; 1.0179x over previous
import jax
import jax.numpy as jnp
from jax.experimental import pallas as pl
from jax.experimental.pallas import tpu as pltpu

EOS = 1e-10
BM = 1024
BN = 1024


def _elu1(a):
    return jnp.where(a > 0, a + 1.0, jnp.exp(a))


def _pair_maps(nb):
    im, jm = [], []
    for i in range(nb):
        for j in range(i, nb):
            im.append(i)
            jm.append(j)
    return jnp.array(im, jnp.int32), jnp.array(jm, jnp.int32)


def _stats_kernel(a_ref, x_ref, w1_ref, b1_ref, rowsum_ref, colpart_ref, h1_ref):
    j = pl.program_id(1)
    e = _elu1(a_ref[:])
    rs = jnp.sum(e, axis=1, keepdims=True)

    @pl.when(j == 0)
    def _():
        rowsum_ref[:] = rs
        h1_ref[:] = (
            jnp.dot(x_ref[:], w1_ref[:], preferred_element_type=jnp.float32)
            + b1_ref[:]
        )

    @pl.when(j != 0)
    def _():
        rowsum_ref[:] += rs

    colpart_ref[:] = jnp.sum(e, axis=0).reshape(1, 1, -1)


def _wait_pair_copies(im_ref, jm_ref, adjn_ref, ab_ref, tb_ref, sema_ref, semb_ref, kk):
    slot = jax.lax.rem(kk, 2)
    i2 = im_ref[kk]
    j2 = jm_ref[kk]
    pltpu.make_async_copy(
        ab_ref.at[slot],
        adjn_ref.at[pl.ds(i2 * BM, BM), pl.ds(j2 * BN, BN)],
        sema_ref.at[slot],
    ).wait()

    @pl.when(i2 != j2)
    def _():
        pltpu.make_async_copy(
            tb_ref.at[slot],
            adjn_ref.at[pl.ds(j2 * BM, BM), pl.ds(i2 * BN, BN)],
            semb_ref.at[slot],
        ).wait()


def _main_kernel(
    im_ref, jm_ref, a_ref, at_ref, h1j_ref, h1i_ref, dc_ref, dr_ref,
    adjn_ref, y1_ref, yp_ref, ab_ref, tb_ref, sema_ref, semb_ref,
):
    k = pl.program_id(0)
    npairs = pl.num_programs(0)
    slot = jax.lax.rem(k, 2)
    i = im_ref[k]
    j = jm_ref[k]

    @pl.when(k >= 2)
    def _():
        _wait_pair_copies(
            im_ref, jm_ref, adjn_ref, ab_ref, tb_ref, sema_ref, semb_ref, k - 2
        )

    e = 0.5 * (_elu1(a_ref[:]) + _elu1(at_ref[:]).T)
    adjn = dc_ref[:] * e * dr_ref[:]
    ab_ref[slot] = adjn
    pltpu.make_async_copy(
        ab_ref.at[slot],
        adjn_ref.at[pl.ds(i * BM, BM), pl.ds(j * BN, BN)],
        sema_ref.at[slot],
    ).start()
    tb_ref[slot] = adjn.T
    c = jnp.dot(adjn, h1j_ref[:], preferred_element_type=jnp.float32)

    @pl.when(i == j)
    def _():
        y1_ref[:] = c
        yp_ref[:] = jnp.zeros_like(yp_ref)

    @pl.when(i != j)
    def _():
        y1_ref[:] += c
        yp_ref[:] = jax.lax.dot_general(
            adjn, h1i_ref[:],
            (((0,), (0,)), ((), ())),
            preferred_element_type=jnp.float32,
        ).reshape(yp_ref.shape)
        pltpu.make_async_copy(
            tb_ref.at[slot],
            adjn_ref.at[pl.ds(j * BM, BM), pl.ds(i * BN, BN)],
            semb_ref.at[slot],
        ).start()

    @pl.when(k == npairs - 1)
    def _():
        @pl.when(k >= 1)
        def _():
            _wait_pair_copies(
                im_ref, jm_ref, adjn_ref, ab_ref, tb_ref, sema_ref, semb_ref, k - 1
            )

        _wait_pair_copies(
            im_ref, jm_ref, adjn_ref, ab_ref, tb_ref, sema_ref, semb_ref, k
        )


def _out_kernel(
    im_ref, jm_ref, adjn_ref, y1j_ref, w2_ref, b2_ref,
    out_ref, op_ref, h2_ref,
):
    k = pl.program_id(0)
    i = im_ref[k]
    j = jm_ref[k]

    @pl.when(i == 0)
    def _():
        h = jnp.maximum(y1j_ref[:], 0.0)
        h2_ref[pl.ds(j * BN, BN), :] = (
            jnp.dot(h, w2_ref[:], preferred_element_type=jnp.float32) + b2_ref[:]
        )

    c = jnp.dot(
        adjn_ref[:], h2_ref[pl.ds(j * BN, BN), :], preferred_element_type=jnp.float32
    )

    @pl.when(i == j)
    def _():
        out_ref[:] = c
        op_ref[:] = jnp.zeros_like(op_ref)

    @pl.when(i != j)
    def _():
        out_ref[:] += c
        op_ref[:] = jax.lax.dot_general(
            adjn_ref[:], h2_ref[pl.ds(i * BM, BM), :],
            (((0,), (0,)), ((), ())),
            preferred_element_type=jnp.float32,
        ).reshape(op_ref.shape)


def kernel(features, x, Adj_param, W1, b1, W2, b2):
    N = Adj_param.shape[0]
    in_dim = x.shape[1]
    hid = W1.shape[1]
    ncls = W2.shape[1]
    nb = N // BM
    npairs = nb * (nb + 1) // 2
    im, jm = _pair_maps(nb)

    rowsum, colpart, h1 = pl.pallas_call(
        _stats_kernel,
        grid=(nb, nb),
        in_specs=[
            pl.BlockSpec((BM, BN), lambda i, j: (i, j)),
            pl.BlockSpec((BM, in_dim), lambda i, j: (i, 0)),
            pl.BlockSpec((in_dim, hid), lambda i, j: (0, 0)),
            pl.BlockSpec((1, hid), lambda i, j: (0, 0)),
        ],
        out_specs=[
            pl.BlockSpec((BM, 1), lambda i, j: (i, 0)),
            pl.BlockSpec((1, 1, BN), lambda i, j: (i, 0, j)),
            pl.BlockSpec((BM, hid), lambda i, j: (i, 0)),
        ],
        out_shape=[
            jax.ShapeDtypeStruct((N, 1), jnp.float32),
            jax.ShapeDtypeStruct((nb, 1, N), jnp.float32),
            jax.ShapeDtypeStruct((N, hid), jnp.float32),
        ],
    )(Adj_param, x, W1, b1.reshape(1, hid))

    deg = 0.5 * (rowsum[:, 0] + jnp.sum(colpart, axis=(0, 1)))
    dinv = 1.0 / (jnp.sqrt(deg) + EOS)
    dc = dinv[:, None]
    dr = dinv[None, :]

    adjn, y1m, y1p = pl.pallas_call(
        _main_kernel,
        grid_spec=pltpu.PrefetchScalarGridSpec(
            num_scalar_prefetch=2,
            grid=(npairs,),
            in_specs=[
                pl.BlockSpec((BM, BN), lambda k, im, jm: (im[k], jm[k])),
                pl.BlockSpec((BN, BM), lambda k, im, jm: (jm[k], im[k])),
                pl.BlockSpec((BN, hid), lambda k, im, jm: (jm[k], 0)),
                pl.BlockSpec((BM, hid), lambda k, im, jm: (im[k], 0)),
                pl.BlockSpec((BM, 1), lambda k, im, jm: (im[k], 0)),
                pl.BlockSpec((1, BN), lambda k, im, jm: (0, jm[k])),
            ],
            out_specs=[
                pl.BlockSpec(memory_space=pl.ANY),
                pl.BlockSpec((BM, hid), lambda k, im, jm: (im[k], 0)),
                pl.BlockSpec((1, BN, hid), lambda k, im, jm: (k, 0, 0)),
            ],
            scratch_shapes=[
                pltpu.VMEM((2, BM, BN), jnp.float32),
                pltpu.VMEM((2, BN, BM), jnp.float32),
                pltpu.SemaphoreType.DMA((2,)),
                pltpu.SemaphoreType.DMA((2,)),
            ],
        ),
        out_shape=[
            jax.ShapeDtypeStruct((N, N), jnp.float32),
            jax.ShapeDtypeStruct((N, hid), jnp.float32),
            jax.ShapeDtypeStruct((npairs, BN, hid), jnp.float32),
        ],
    )(im, jm, Adj_param, Adj_param, h1, h1, dc, dr)

    y1 = y1m + jax.ops.segment_sum(
        y1p, jm, num_segments=nb, indices_are_sorted=True
    ).reshape(N, hid)

    out_m, out_p = pl.pallas_call(
        _out_kernel,
        grid_spec=pltpu.PrefetchScalarGridSpec(
            num_scalar_prefetch=2,
            grid=(npairs,),
            in_specs=[
                pl.BlockSpec((BM, BN), lambda k, im, jm: (im[k], jm[k])),
                pl.BlockSpec((BN, hid), lambda k, im, jm: (jm[k], 0)),
                pl.BlockSpec((hid, ncls), lambda k, im, jm: (0, 0)),
                pl.BlockSpec((1, ncls), lambda k, im, jm: (0, 0)),
            ],
            out_specs=[
                pl.BlockSpec((BM, ncls), lambda k, im, jm: (im[k], 0)),
                pl.BlockSpec((1, BN, ncls), lambda k, im, jm: (k, 0, 0)),
            ],
            scratch_shapes=[pltpu.VMEM((N, ncls), jnp.float32)],
        ),
        out_shape=[
            jax.ShapeDtypeStruct((N, ncls), jnp.float32),
            jax.ShapeDtypeStruct((npairs, BN, ncls), jnp.float32),
        ],
    )(im, jm, adjn, y1, W2, b2.reshape(1, ncls))

    out = out_m + jax.ops.segment_sum(
        out_p, jm, num_segments=nb, indices_are_sorted=True
    ).reshape(N, ncls)

    return (out, adjn)


# speedup vs baseline: 1.4991x; 1.2053x over previous
import jax
import jax.numpy as jnp
from jax.experimental import pallas as pl
from jax.experimental.pallas import tpu as pltpu

EOS = 1e-10
BM = 1024
BN = 1024


def _elu1(a):
    return jnp.where(a > 0, a + 1.0, jnp.exp(a))


def _pair_maps(nb):
    im, jm = [], []
    for i in range(nb):
        for j in range(i, nb):
            im.append(i)
            jm.append(j)
    return jnp.array(im, jnp.int32), jnp.array(jm, jnp.int32)


def _stats_kernel(a_ref, x_ref, w1_ref, b1_ref, rowsum_ref, colpart_ref, h1_ref):
    j = pl.program_id(1)
    e = _elu1(a_ref[:])
    rs = jnp.sum(e, axis=1, keepdims=True)

    @pl.when(j == 0)
    def _():
        rowsum_ref[:] = rs
        h1_ref[:] = (
            jnp.dot(x_ref[:], w1_ref[:], preferred_element_type=jnp.float32)
            + b1_ref[:]
        )

    @pl.when(j != 0)
    def _():
        rowsum_ref[:] += rs

    colpart_ref[:] = jnp.sum(e, axis=0).reshape(1, 1, -1)


def _wait_pair_copies(im_ref, jm_ref, adjn_ref, ab_ref, tb_ref, sema_ref, semb_ref, kk):
    slot = jax.lax.rem(kk, 2)
    i2 = im_ref[kk]
    j2 = jm_ref[kk]
    pltpu.make_async_copy(
        ab_ref.at[slot],
        adjn_ref.at[pl.ds(i2 * BM, BM), pl.ds(j2 * BN, BN)],
        sema_ref.at[slot],
    ).wait()

    @pl.when(i2 != j2)
    def _():
        pltpu.make_async_copy(
            tb_ref.at[slot],
            adjn_ref.at[pl.ds(j2 * BM, BM), pl.ds(i2 * BN, BN)],
            semb_ref.at[slot],
        ).wait()


def _main_kernel(
    im_ref, jm_ref, a_ref, at_ref, h1j_ref, h1i_ref, dc_ref, dr_ref,
    adjn_ref, y1_ref, ab_ref, tb_ref, sema_ref, semb_ref,
):
    k = pl.program_id(0)
    npairs = pl.num_programs(0)
    slot = jax.lax.rem(k, 2)
    i = im_ref[k]
    j = jm_ref[k]

    @pl.when(k >= 2)
    def _():
        _wait_pair_copies(
            im_ref, jm_ref, adjn_ref, ab_ref, tb_ref, sema_ref, semb_ref, k - 2
        )

    @pl.when(k == 0)
    def _():
        y1_ref[:] = jnp.zeros_like(y1_ref)

    e = 0.5 * (_elu1(a_ref[:]) + _elu1(at_ref[:]).T)
    adjn = dc_ref[:] * e * dr_ref[:]
    ab_ref[slot] = adjn
    pltpu.make_async_copy(
        ab_ref.at[slot],
        adjn_ref.at[pl.ds(i * BM, BM), pl.ds(j * BN, BN)],
        sema_ref.at[slot],
    ).start()
    tb_ref[slot] = adjn.T
    y1_ref[pl.ds(i * BM, BM), :] += jnp.dot(
        adjn, h1j_ref[:], preferred_element_type=jnp.float32
    )

    @pl.when(i != j)
    def _():
        y1_ref[pl.ds(j * BN, BN), :] += jax.lax.dot_general(
            adjn, h1i_ref[:],
            (((0,), (0,)), ((), ())),
            preferred_element_type=jnp.float32,
        )
        pltpu.make_async_copy(
            tb_ref.at[slot],
            adjn_ref.at[pl.ds(j * BM, BM), pl.ds(i * BN, BN)],
            semb_ref.at[slot],
        ).start()

    @pl.when(k == npairs - 1)
    def _():
        @pl.when(k >= 1)
        def _():
            _wait_pair_copies(
                im_ref, jm_ref, adjn_ref, ab_ref, tb_ref, sema_ref, semb_ref, k - 1
            )

        _wait_pair_copies(
            im_ref, jm_ref, adjn_ref, ab_ref, tb_ref, sema_ref, semb_ref, k
        )


def _out_kernel(
    im_ref, jm_ref, adjn_ref, y1j_ref, w2_ref, b2_ref,
    out_ref, h2_ref,
):
    k = pl.program_id(0)
    i = im_ref[k]
    j = jm_ref[k]

    @pl.when(k == 0)
    def _():
        out_ref[:] = jnp.zeros_like(out_ref)

    @pl.when(i == 0)
    def _():
        h = jnp.maximum(y1j_ref[:], 0.0)
        h2_ref[pl.ds(j * BN, BN), :] = (
            jnp.dot(h, w2_ref[:], preferred_element_type=jnp.float32) + b2_ref[:]
        )

    out_ref[pl.ds(i * BM, BM), :] += jnp.dot(
        adjn_ref[:], h2_ref[pl.ds(j * BN, BN), :], preferred_element_type=jnp.float32
    )

    @pl.when(i != j)
    def _():
        out_ref[pl.ds(j * BN, BN), :] += jax.lax.dot_general(
            adjn_ref[:], h2_ref[pl.ds(i * BM, BM), :],
            (((0,), (0,)), ((), ())),
            preferred_element_type=jnp.float32,
        )


def kernel(features, x, Adj_param, W1, b1, W2, b2):
    N = Adj_param.shape[0]
    in_dim = x.shape[1]
    hid = W1.shape[1]
    ncls = W2.shape[1]
    nb = N // BM
    npairs = nb * (nb + 1) // 2
    im, jm = _pair_maps(nb)

    rowsum, colpart, h1 = pl.pallas_call(
        _stats_kernel,
        grid=(nb, nb),
        in_specs=[
            pl.BlockSpec((BM, BN), lambda i, j: (i, j)),
            pl.BlockSpec((BM, in_dim), lambda i, j: (i, 0)),
            pl.BlockSpec((in_dim, hid), lambda i, j: (0, 0)),
            pl.BlockSpec((1, hid), lambda i, j: (0, 0)),
        ],
        out_specs=[
            pl.BlockSpec((BM, 1), lambda i, j: (i, 0)),
            pl.BlockSpec((1, 1, BN), lambda i, j: (i, 0, j)),
            pl.BlockSpec((BM, hid), lambda i, j: (i, 0)),
        ],
        out_shape=[
            jax.ShapeDtypeStruct((N, 1), jnp.float32),
            jax.ShapeDtypeStruct((nb, 1, N), jnp.float32),
            jax.ShapeDtypeStruct((N, hid), jnp.float32),
        ],
    )(Adj_param, x, W1, b1.reshape(1, hid))

    deg = 0.5 * (rowsum[:, 0] + jnp.sum(colpart, axis=(0, 1)))
    dinv = 1.0 / (jnp.sqrt(deg) + EOS)
    dc = dinv[:, None]
    dr = dinv[None, :]

    adjn, y1 = pl.pallas_call(
        _main_kernel,
        grid_spec=pltpu.PrefetchScalarGridSpec(
            num_scalar_prefetch=2,
            grid=(npairs,),
            in_specs=[
                pl.BlockSpec((BM, BN), lambda k, im, jm: (im[k], jm[k])),
                pl.BlockSpec((BN, BM), lambda k, im, jm: (jm[k], im[k])),
                pl.BlockSpec((BN, hid), lambda k, im, jm: (jm[k], 0)),
                pl.BlockSpec((BM, hid), lambda k, im, jm: (im[k], 0)),
                pl.BlockSpec((BM, 1), lambda k, im, jm: (im[k], 0)),
                pl.BlockSpec((1, BN), lambda k, im, jm: (0, jm[k])),
            ],
            out_specs=[
                pl.BlockSpec(memory_space=pl.ANY),
                pl.BlockSpec((N, hid), lambda k, im, jm: (0, 0)),
            ],
            scratch_shapes=[
                pltpu.VMEM((2, BM, BN), jnp.float32),
                pltpu.VMEM((2, BN, BM), jnp.float32),
                pltpu.SemaphoreType.DMA((2,)),
                pltpu.SemaphoreType.DMA((2,)),
            ],
        ),
        out_shape=[
            jax.ShapeDtypeStruct((N, N), jnp.float32),
            jax.ShapeDtypeStruct((N, hid), jnp.float32),
        ],
    )(im, jm, Adj_param, Adj_param, h1, h1, dc, dr)

    out = pl.pallas_call(
        _out_kernel,
        grid_spec=pltpu.PrefetchScalarGridSpec(
            num_scalar_prefetch=2,
            grid=(npairs,),
            in_specs=[
                pl.BlockSpec((BM, BN), lambda k, im, jm: (im[k], jm[k])),
                pl.BlockSpec((BN, hid), lambda k, im, jm: (jm[k], 0)),
                pl.BlockSpec((hid, ncls), lambda k, im, jm: (0, 0)),
                pl.BlockSpec((1, ncls), lambda k, im, jm: (0, 0)),
            ],
            out_specs=pl.BlockSpec((N, ncls), lambda k, im, jm: (0, 0)),
            scratch_shapes=[pltpu.VMEM((N, ncls), jnp.float32)],
        ),
        out_shape=jax.ShapeDtypeStruct((N, ncls), jnp.float32),
    )(im, jm, adjn, y1, W2, b2.reshape(1, ncls))

    return (out, adjn)


# speedup vs baseline: 1.5173x; 1.0122x over previous
import jax
import jax.numpy as jnp
from jax.experimental import pallas as pl
from jax.experimental.pallas import tpu as pltpu

EOS = 1e-10
BM = 1024
BN = 1024


def _elu1(a):
    return jnp.where(a > 0, a + 1.0, jnp.exp(a))


def _pair_maps(nb):
    im, jm = [], []
    for i in range(nb):
        for j in range(i, nb):
            im.append(i)
            jm.append(j)
    return jnp.array(im, jnp.int32), jnp.array(jm, jnp.int32)


def _pair_maps_main(nb):
    im, jm, ati, atj = [], [], [], []
    prev = None
    for i in range(nb):
        for (a, b) in [(i, j) for j in range(i + 1, nb)] + [(i, i)]:
            im.append(a)
            jm.append(b)
            cur = (b, a) if a != b else (prev if prev is not None else (a, b))
            ati.append(cur[0])
            atj.append(cur[1])
            prev = cur
    return (jnp.array(im, jnp.int32), jnp.array(jm, jnp.int32),
            jnp.array(ati, jnp.int32), jnp.array(atj, jnp.int32))


def _stats_kernel(a_ref, x_ref, w1_ref, b1_ref, rowsum_ref, colpart_ref, h1_ref):
    j = pl.program_id(1)
    e = _elu1(a_ref[:])
    rs = jnp.sum(e, axis=1, keepdims=True)

    @pl.when(j == 0)
    def _():
        rowsum_ref[:] = rs
        h1_ref[:] = (
            jnp.dot(x_ref[:], w1_ref[:], preferred_element_type=jnp.float32)
            + b1_ref[:]
        )

    @pl.when(j != 0)
    def _():
        rowsum_ref[:] += rs

    colpart_ref[:] = jnp.sum(e, axis=0).reshape(1, 1, -1)


def _wait_pair_copies(im_ref, jm_ref, adjn_ref, ab_ref, tb_ref, sema_ref, semb_ref, kk):
    slot = jax.lax.rem(kk, 2)
    i2 = im_ref[kk]
    j2 = jm_ref[kk]
    pltpu.make_async_copy(
        ab_ref.at[slot],
        adjn_ref.at[pl.ds(i2 * BM, BM), pl.ds(j2 * BN, BN)],
        sema_ref.at[slot],
    ).wait()

    @pl.when(i2 != j2)
    def _():
        pltpu.make_async_copy(
            tb_ref.at[slot],
            adjn_ref.at[pl.ds(j2 * BM, BM), pl.ds(i2 * BN, BN)],
            semb_ref.at[slot],
        ).wait()


def _main_kernel(
    im_ref, jm_ref, ati_ref, atj_ref,
    a_ref, at_ref, h1j_ref, h1i_ref, dc_ref, dr_ref,
    adjn_ref, y1_ref, ab_ref, tb_ref, sema_ref, semb_ref,
):
    k = pl.program_id(0)
    npairs = pl.num_programs(0)
    slot = jax.lax.rem(k, 2)
    i = im_ref[k]
    j = jm_ref[k]

    @pl.when(k >= 2)
    def _():
        _wait_pair_copies(
            im_ref, jm_ref, adjn_ref, ab_ref, tb_ref, sema_ref, semb_ref, k - 2
        )

    @pl.when(k == 0)
    def _():
        y1_ref[:] = jnp.zeros_like(y1_ref)

    t = jnp.where(i == j, a_ref[:], at_ref[:])
    e = 0.5 * (_elu1(a_ref[:]) + _elu1(t).T)
    adjn = dc_ref[:] * e * dr_ref[:]
    ab_ref[slot] = adjn
    pltpu.make_async_copy(
        ab_ref.at[slot],
        adjn_ref.at[pl.ds(i * BM, BM), pl.ds(j * BN, BN)],
        sema_ref.at[slot],
    ).start()
    tb_ref[slot] = adjn.T
    y1_ref[pl.ds(i * BM, BM), :] += jnp.dot(
        adjn, h1j_ref[:], preferred_element_type=jnp.float32
    )

    @pl.when(i != j)
    def _():
        y1_ref[pl.ds(j * BN, BN), :] += jax.lax.dot_general(
            adjn, h1i_ref[:],
            (((0,), (0,)), ((), ())),
            preferred_element_type=jnp.float32,
        )
        pltpu.make_async_copy(
            tb_ref.at[slot],
            adjn_ref.at[pl.ds(j * BM, BM), pl.ds(i * BN, BN)],
            semb_ref.at[slot],
        ).start()

    @pl.when(k == npairs - 1)
    def _():
        @pl.when(k >= 1)
        def _():
            _wait_pair_copies(
                im_ref, jm_ref, adjn_ref, ab_ref, tb_ref, sema_ref, semb_ref, k - 1
            )

        _wait_pair_copies(
            im_ref, jm_ref, adjn_ref, ab_ref, tb_ref, sema_ref, semb_ref, k
        )


def _out_kernel(
    im_ref, jm_ref, ym_ref, adjn_ref, y1j_ref, w2_ref, b2_ref,
    out_ref, h2_ref,
):
    k = pl.program_id(0)
    i = im_ref[k]
    j = jm_ref[k]

    @pl.when(k == 0)
    def _():
        out_ref[:] = jnp.zeros_like(out_ref)

    @pl.when(i == 0)
    def _():
        h = jnp.maximum(y1j_ref[:], 0.0)
        h2_ref[pl.ds(j * BN, BN), :] = (
            jnp.dot(h, w2_ref[:], preferred_element_type=jnp.float32) + b2_ref[:]
        )

    out_ref[pl.ds(i * BM, BM), :] += jnp.dot(
        adjn_ref[:], h2_ref[pl.ds(j * BN, BN), :], preferred_element_type=jnp.float32
    )

    @pl.when(i != j)
    def _():
        out_ref[pl.ds(j * BN, BN), :] += jax.lax.dot_general(
            adjn_ref[:], h2_ref[pl.ds(i * BM, BM), :],
            (((0,), (0,)), ((), ())),
            preferred_element_type=jnp.float32,
        )


def kernel(features, x, Adj_param, W1, b1, W2, b2):
    N = Adj_param.shape[0]
    in_dim = x.shape[1]
    hid = W1.shape[1]
    ncls = W2.shape[1]
    nb = N // BM
    npairs = nb * (nb + 1) // 2
    im, jm = _pair_maps(nb)

    rowsum, colpart, h1 = pl.pallas_call(
        _stats_kernel,
        grid=(nb, nb),
        in_specs=[
            pl.BlockSpec((BM, BN), lambda i, j: (i, j)),
            pl.BlockSpec((BM, in_dim), lambda i, j: (i, 0)),
            pl.BlockSpec((in_dim, hid), lambda i, j: (0, 0)),
            pl.BlockSpec((1, hid), lambda i, j: (0, 0)),
        ],
        out_specs=[
            pl.BlockSpec((BM, 1), lambda i, j: (i, 0)),
            pl.BlockSpec((1, 1, BN), lambda i, j: (i, 0, j)),
            pl.BlockSpec((BM, hid), lambda i, j: (i, 0)),
        ],
        out_shape=[
            jax.ShapeDtypeStruct((N, 1), jnp.float32),
            jax.ShapeDtypeStruct((nb, 1, N), jnp.float32),
            jax.ShapeDtypeStruct((N, hid), jnp.float32),
        ],
    )(Adj_param, x, W1, b1.reshape(1, hid))

    deg = 0.5 * (rowsum[:, 0] + jnp.sum(colpart, axis=(0, 1)))
    dinv = 1.0 / (jnp.sqrt(deg) + EOS)
    dc = dinv[:, None]
    dr = dinv[None, :]

    imm, jmm, ati, atj = _pair_maps_main(nb)
    adjn, y1 = pl.pallas_call(
        _main_kernel,
        grid_spec=pltpu.PrefetchScalarGridSpec(
            num_scalar_prefetch=4,
            grid=(npairs,),
            in_specs=[
                pl.BlockSpec((BM, BN), lambda k, im, jm, ai, aj: (im[k], jm[k])),
                pl.BlockSpec((BN, BM), lambda k, im, jm, ai, aj: (ai[k], aj[k])),
                pl.BlockSpec((BN, hid), lambda k, im, jm, ai, aj: (jm[k], 0)),
                pl.BlockSpec((BM, hid), lambda k, im, jm, ai, aj: (im[k], 0)),
                pl.BlockSpec((BM, 1), lambda k, im, jm, ai, aj: (im[k], 0)),
                pl.BlockSpec((1, BN), lambda k, im, jm, ai, aj: (0, jm[k])),
            ],
            out_specs=[
                pl.BlockSpec(memory_space=pl.ANY),
                pl.BlockSpec((N, hid), lambda k, im, jm, ai, aj: (0, 0)),
            ],
            scratch_shapes=[
                pltpu.VMEM((2, BM, BN), jnp.float32),
                pltpu.VMEM((2, BN, BM), jnp.float32),
                pltpu.SemaphoreType.DMA((2,)),
                pltpu.SemaphoreType.DMA((2,)),
            ],
        ),
        out_shape=[
            jax.ShapeDtypeStruct((N, N), jnp.float32),
            jax.ShapeDtypeStruct((N, hid), jnp.float32),
        ],
    )(imm, jmm, ati, atj, Adj_param, Adj_param, h1, h1, dc, dr)

    ym = jnp.where(im == 0, jm, nb - 1)
    out = pl.pallas_call(
        _out_kernel,
        grid_spec=pltpu.PrefetchScalarGridSpec(
            num_scalar_prefetch=3,
            grid=(npairs,),
            in_specs=[
                pl.BlockSpec((BM, BN), lambda k, im, jm, ym: (im[k], jm[k])),
                pl.BlockSpec((BN, hid), lambda k, im, jm, ym: (ym[k], 0)),
                pl.BlockSpec((hid, ncls), lambda k, im, jm, ym: (0, 0)),
                pl.BlockSpec((1, ncls), lambda k, im, jm, ym: (0, 0)),
            ],
            out_specs=pl.BlockSpec((N, ncls), lambda k, im, jm, ym: (0, 0)),
            scratch_shapes=[pltpu.VMEM((N, ncls), jnp.float32)],
        ),
        out_shape=jax.ShapeDtypeStruct((N, ncls), jnp.float32),
    )(im, jm, ym, adjn, y1, W2, b2.reshape(1, ncls))

    return (out, adjn)
